# baseline (speedup 1.0000x reference)
import jax
import jax.numpy as jnp
from jax import lax
from jax.experimental import pallas as pl
from jax.experimental.pallas import tpu as pltpu
from jax.experimental.pallas import tpu_sc as plsc

NU = 50000
D = 64
DH = 32
E_RATE = 800000
E_TRUST = 800000
E_PRED = 200000
LAMDA = 0.5
LAMDA_T = 0.25

NC, NS, L = 2, 16, 16
HALF = NU // NC
TRASH = 512
ACC_ROWS = 25600
CR, CC = 5, 128
CB = CR * CC
NCH_E = E_RATE // CB
KMAX_E = 80
NCHP_E = KMAX_E * NS
CRP = 5
NCH_P = 2 * E_PRED // (CRP * CC)
KMAX_P = 20
NCHP_P = KMAX_P * NC * NS
EXTW = 80
WFULL = 1568
WTAIL = HALF - (NS - 1) * WFULL

_mesh = plsc.VectorSubcoreMesh(core_axis_name="c", subcore_axis_name="s",
                               num_cores=NC, num_subcores=NS)
_sc_params = pltpu.CompilerParams(use_tc_tiling_on_sc=False)


def _lanesum(v):
  iota = lax.iota(jnp.int32, L)
  for sh in (8, 4, 2, 1):
    v = v + v.at[jnp.bitwise_xor(iota, sh)].get(mode="promise_in_bounds")
  return v


def _make_edge_pass(do_dots):
  out_type = [
      jax.ShapeDtypeStruct((2, NU, DH), jnp.float32),
      jax.ShapeDtypeStruct((NU,), jnp.float32),
      jax.ShapeDtypeStruct((NU,), jnp.float32),
  ]
  scratch = [
      pltpu.VMEM((2, CR, CC), jnp.int32),
      pltpu.VMEM((2, CR, CC), jnp.int32),
      pltpu.VMEM((2, CR, CC), jnp.int32),
      pltpu.VMEM((2, CR, CC), jnp.int32),
      pltpu.VMEM((2, CR, CC), jnp.int32),
      pltpu.VMEM((2, CR, CC, DH), jnp.float32),
      pltpu.VMEM((CR, CC), jnp.float32),
      pltpu.VMEM((64, DH), jnp.float32),
      pltpu.VMEM((400,), jnp.float32),
      pltpu.SemaphoreType.DMA,
      pltpu.SemaphoreType.DMA,
      pltpu.VMEM_SHARED((ACC_ROWS, DH), jnp.float32),
      pltpu.VMEM_SHARED((ACC_ROWS,), jnp.float32),
      pltpu.VMEM_SHARED((ACC_ROWS,), jnp.float32),
  ]
  if do_dots:
    out_type.append(jax.ShapeDtypeStruct((NC * NS, 2, L), jnp.float32))
    out_type.append(jax.ShapeDtypeStruct((NCH_E, CR, CC), jnp.float32))
    scratch.append(pltpu.VMEM((CC, DH), jnp.float32))
    scratch.append(pltpu.VMEM((2, L), jnp.float32))
    scratch.append(pltpu.VMEM((CR, CC), jnp.float32))

  def body(*refs):
    if do_dots:
      (idx_g, idx_s, gtab, ptab, sums_o, ha_o, hb_o, link_o, dbuf,
       ib_g, ib_s, ib_x, lb_s, lb_g, rows, ones, zb, zv, gsem, ssem,
       acc, ha, hb, prows, sb, db) = refs
    else:
      (idx_g, idx_s, gtab, sums_o, ha_o, hb_o,
       ib_g, ib_s, ib_x, lb_s, lb_g, rows, ones, zb, zv, gsem, ssem,
       acc, ha, hb) = refs
    c = lax.axis_index("c")
    s = lax.axis_index("s")
    iota = lax.iota(jnp.int32, L)

    def zrow(r, u):
      for j in range(DH // L):
        zb[r, pl.ds(j * L, L)] = jnp.zeros((L,), jnp.float32)
      return u
    lax.fori_loop(0, 64, zrow, 0)

    def zvec(r, u):
      zv[pl.ds(r * L, L)] = jnp.zeros((L,), jnp.float32)
      return u
    lax.fori_loop(0, 400 // L, zvec, 0)

    def orow(r, u):
      for j in range(CC // L):
        ones[r, pl.ds(j * L, L)] = jnp.ones((L,), jnp.float32)
      return u
    lax.fori_loop(0, CR, orow, 0)
    if do_dots:
      sb[0, :] = jnp.zeros((L,), jnp.float32)
      sb[1, :] = jnp.zeros((L,), jnp.float32)

    for h in (0, 1):
      def zacc(r, u):
        pltpu.sync_copy(zb, acc.at[pl.ds(s * 1600 + r * 64, 64), :])
        return u
      lax.fori_loop(0, 1600 // 64, zacc, 0)
      if h == 0:
        def zhist(r, u):
          pltpu.sync_copy(zv, ha.at[pl.ds(s * 1600 + r * 400, 400)])
          pltpu.sync_copy(zv, hb.at[pl.ds(s * 1600 + r * 400, 400)])
          return u
        lax.fori_loop(0, 4, zhist, 0)
      plsc.subcore_barrier()

      def kof(t):
        return jnp.minimum(t * NS + s, NCHP_E - 1)

      def load_idx(t, b):
        k = kof(t)
        pltpu.sync_copy(idx_g.at[k], ib_g.at[b])
        pltpu.sync_copy(idx_s.at[k], ib_s.at[b])
        for a in range(CR):
          for j in range(CC // L):
            sl = pl.ds(j * L, L)
            tb = lax.rem(t * CC + (a * (CC // L) + j) * L + s * L,
                         jnp.int32(TRASH))
            tv = HALF + tb + iota
            sv = ib_s[b, a, sl]
            lvs = sv - c * HALF
            oks = (lvs >= 0) & (lvs < HALF)
            lb_s[b, a, sl] = jnp.where(oks, lvs, tv)
            gv = ib_g[b, a, sl]
            ib_x[b, a, sl] = jnp.minimum(gv, NU - 1) * 2 + h
            if h == 0:
              lvg = gv - c * HALF
              okg = (lvg >= 0) & (lvg < HALF)
              lb_g[b, a, sl] = jnp.where(okg, lvg, tv)

      def fire_g(b):
        for a in range(CR):
          pltpu.async_copy(gtab.at[ib_x.at[b, a]], rows.at[b, a], gsem)

      def drain_g(b):
        for a in range(CR):
          pltpu.make_async_copy(gtab.at[ib_x.at[b, a]], rows.at[b, a],
                                gsem).wait()

      def fire_s(b):
        for a in range(CR):
          pltpu.async_copy(rows.at[b, a], acc.at[lb_s.at[b, a]], ssem,
                           add=True)
        if h == 0:
          for a in range(CR):
            pltpu.async_copy(ones.at[a], ha.at[lb_s.at[b, a]], ssem,
                             add=True)
            pltpu.async_copy(ones.at[a], hb.at[lb_g.at[b, a]], ssem,
                             add=True)

      def drain_s(b):
        for a in range(CR):
          pltpu.make_async_copy(rows.at[b, a], acc.at[lb_s.at[b, a]],
                                ssem).wait()
        if h == 0:
          for a in range(CR):
            pltpu.make_async_copy(ones.at[a], ha.at[lb_s.at[b, a]],
                                  ssem).wait()
            pltpu.make_async_copy(ones.at[a], hb.at[lb_g.at[b, a]],
                                  ssem).wait()

      def dots(t, b):
        if not do_dots:
          return
        k = kof(t)
        duty = (lax.rem(t, jnp.int32(2)) == c) & (k < NCH_E)

        @pl.when(duty)
        def _():
          for a in range(CR):
            for j in range(CC // L):
              sl = pl.ds(j * L, L)
              ib_x[b, a, sl] = ib_s[b, a, sl] * 2 + h
          if h == 1:
            pltpu.sync_copy(dbuf.at[k], db)
          for a in range(CR):
            pltpu.sync_copy(ptab.at[ib_x.at[b, a]], prows)

            def g_body(g, u2):
              dvec = jnp.zeros((L,), jnp.float32)
              for ee in range(L):
                e = g * L + ee
                pr = jnp.zeros((L,), jnp.float32)
                for j in range(DH // L):
                  pr = pr + (rows[b, a, e, pl.ds(j * L, L)]
                             * prows[e, pl.ds(j * L, L)])
                d = _lanesum(pr)
                dvec = jnp.where(iota == ee, d, dvec)
              sl = pl.ds(g * L, L)
              if h == 0:
                db[a, sl] = dvec
              else:
                tot = db[a, sl] + dvec
                sb[0, :] = sb[0, :] + tot
                sb[1, :] = sb[1, :] + tot * tot
              return u2
            lax.fori_loop(0, CC // L, g_body, 0)
          if h == 0:
            pltpu.sync_copy(db, dbuf.at[k])

      load_idx(jnp.int32(0), 0)
      fire_g(0)

      def pair_body(q, u):
        t0 = 2 * q
        load_idx(t0 + 1, 1)
        drain_g(0)
        fire_s(0)
        fire_g(1)
        dots(t0, 0)
        drain_s(0)
        load_idx(t0 + 2, 0)
        fire_g(0)
        drain_g(1)
        fire_s(1)
        dots(t0 + 1, 1)
        drain_s(1)
        return u

      lax.fori_loop(0, KMAX_E // 2, pair_body, 0)
      drain_g(0)
      plsc.subcore_barrier()

      @pl.when(s < NS - 1)
      def _():
        pltpu.sync_copy(acc.at[pl.ds(s * WFULL, WFULL), :],
                        sums_o.at[h, pl.ds(c * HALF + s * WFULL, WFULL), :])
        if h == 0:
          pltpu.sync_copy(ha.at[pl.ds(s * WFULL, WFULL)],
                          ha_o.at[pl.ds(c * HALF + s * WFULL, WFULL)])
          pltpu.sync_copy(hb.at[pl.ds(s * WFULL, WFULL)],
                          hb_o.at[pl.ds(c * HALF + s * WFULL, WFULL)])

      @pl.when(s == NS - 1)
      def _():
        base = (NS - 1) * WFULL
        pltpu.sync_copy(acc.at[pl.ds(base, WTAIL), :],
                        sums_o.at[h, pl.ds(c * HALF + base, WTAIL), :])
        if h == 0:
          pltpu.sync_copy(ha.at[pl.ds(base, WTAIL)],
                          ha_o.at[pl.ds(c * HALF + base, WTAIL)])
          pltpu.sync_copy(hb.at[pl.ds(base, WTAIL)],
                          hb_o.at[pl.ds(c * HALF + base, WTAIL)])
      plsc.subcore_barrier()

    if do_dots:
      pltpu.sync_copy(sb, link_o.at[c * NS + s])

  return pl.kernel(body, out_type=tuple(out_type), mesh=_mesh,
                   compiler_params=_sc_params, scratch_types=tuple(scratch))


_edge_nodot = _make_edge_pass(False)
_edge_dot = _make_edge_pass(True)


def _score_body(idx_u, idx_i, uext, iext, out,
                ib_u, ib_i, urows, irows, scoreb, gsem):
  c = lax.axis_index("c")
  s = lax.axis_index("s")
  wid = c * NS + s
  iota = lax.iota(jnp.int32, L)

  def fire(b, a):
    pltpu.async_copy(uext.at[ib_u.at[a]], urows.at[b], gsem)
    pltpu.async_copy(iext.at[ib_i.at[a]], irows.at[b], gsem)

  def drain(b, a):
    pltpu.make_async_copy(uext.at[ib_u.at[a]], urows.at[b], gsem).wait()
    pltpu.make_async_copy(iext.at[ib_i.at[a]], irows.at[b], gsem).wait()

  def dot_slab(b, a):
    def gg_body(g, u2):
      outv = jnp.zeros((L,), jnp.float32)
      for ee in range(L):
        e = g * L + ee
        pr = jnp.zeros((L,), jnp.float32)
        for j in range(EXTW // L):
          pr = pr + (urows[b, e, pl.ds(j * L, L)]
                     * irows[b, e, pl.ds(j * L, L)])
        d = _lanesum(pr)
        outv = jnp.where(iota == ee, d, outv)
      scoreb[a, pl.ds(g * L, L)] = outv
      return u2
    lax.fori_loop(0, CC // L, gg_body, 0)

  def round_body(t, u):
    k = t * NC * NS + wid
    pltpu.sync_copy(idx_u.at[k], ib_u)
    pltpu.sync_copy(idx_i.at[k], ib_i)
    fire(0, 0)
    for a in range(CRP):
      drain(a % 2, a)
      if a + 1 < CRP:
        fire((a + 1) % 2, a + 1)
      dot_slab(a % 2, a)
    pltpu.sync_copy(scoreb, out.at[k])
    return u
  lax.fori_loop(0, KMAX_P, round_body, 0)


_score = pl.kernel(
    _score_body,
    out_type=(jax.ShapeDtypeStruct((NCHP_P, CRP, CC), jnp.float32),),
    mesh=_mesh,
    compiler_params=_sc_params,
    scratch_types=(
        pltpu.VMEM((CRP, CC), jnp.int32),
        pltpu.VMEM((CRP, CC), jnp.int32),
        pltpu.VMEM((2, CC, EXTW), jnp.float32),
        pltpu.VMEM((2, CC, EXTW), jnp.float32),
        pltpu.VMEM((CRP, CC), jnp.float32),
        pltpu.SemaphoreType.DMA,
    ))

_R = 1000
_G = NU // _R


def _dense_body(sy0, sy1, sw0, sw1, pqu, pqi, ywi, ywu, bu, bi,
                dru, dri, dti, dto, lp, gb, ue, ie, reg, link):
  i = pl.program_id(0)

  def fac(d):
    return jnp.where(d > 0, lax.rsqrt(jnp.maximum(d, 1.0)), 0.0)

  I_f = fac(dru[...])
  T_f = fac(dti[...])
  U_j = fac(dri[...])
  T_v = fac(dto[...])
  syr = jnp.concatenate([sy0[...], sy1[...]], axis=1)
  swr = jnp.concatenate([sw0[...], sw1[...]], axis=1)
  res = T_f * swr + I_f * syr + pqu[...]
  one = jnp.ones((_R, 1), jnp.float32)
  zer = jnp.zeros((_R, EXTW - D - 3), jnp.float32)
  gbv = jnp.full((_R, 1), gb[0, 0], jnp.float32)
  ue[...] = jnp.concatenate([res, bu[...], one, one, zer], axis=1)
  ie[...] = jnp.concatenate([pqi[...], one, bi[...], gbv, zer], axis=1)
  part = (LAMDA * jnp.sum(I_f * bu[...] ** 2)
          + LAMDA * jnp.sum(U_j * bi[...] ** 2)
          + jnp.sum((LAMDA * I_f + LAMDA_T * T_f)
                    * jnp.sum(pqu[...] ** 2, axis=1, keepdims=True))
          + LAMDA * jnp.sum(U_j * jnp.sum(pqi[...] ** 2, axis=1,
                                          keepdims=True))
          + LAMDA * jnp.sum(U_j * jnp.sum(ywi[...] ** 2, axis=1,
                                          keepdims=True))
          + LAMDA_T * jnp.sum(T_v * jnp.sum(ywu[...] ** 2, axis=1,
                                            keepdims=True)))
  prev = jnp.where(i == 0, jnp.zeros((1, 1), jnp.float32), reg[...])
  tot = prev + part
  reg[...] = jnp.where(i == _G - 1, tot / NU, tot)

  @pl.when(i == 0)
  def _():
    lpv = lp[...]
    sd = jnp.sum(lpv[:, :L])
    sd2 = jnp.sum(lpv[:, L:]) / L
    link[...] = jnp.full((1, 1), LAMDA_T * (sd2 - 2.0 * sd + E_TRUST)
                         / E_TRUST, jnp.float32)


_dense = pl.pallas_call(
    _dense_body,
    grid=(_G,),
    in_specs=[pl.BlockSpec((_R, DH), lambda i: (i, 0))] * 4
    + [pl.BlockSpec((_R, D), lambda i: (i, 0))] * 4
    + [pl.BlockSpec((_R, 1), lambda i: (i, 0))] * 6
    + [pl.BlockSpec((NC * NS, 2 * L), lambda i: (0, 0)),
       pl.BlockSpec((1, 1), lambda i: (0, 0))],
    out_specs=[pl.BlockSpec((_R, EXTW), lambda i: (i, 0)),
               pl.BlockSpec((_R, EXTW), lambda i: (i, 0)),
               pl.BlockSpec((1, 1), lambda i: (0, 0)),
               pl.BlockSpec((1, 1), lambda i: (0, 0))],
    out_shape=[jax.ShapeDtypeStruct((NU, EXTW), jnp.float32),
               jax.ShapeDtypeStruct((NU, EXTW), jnp.float32),
               jax.ShapeDtypeStruct((1, 1), jnp.float32),
               jax.ShapeDtypeStruct((1, 1), jnp.float32)],
)


def _pad_edges(v, total):
  return jnp.concatenate([v, jnp.full((total - v.shape[0],), NU, jnp.int32)])


def kernel(rate_edge_index, trust_edge_index, pos_edge_index, neg_edge_index,
           p_q_user, p_q_item, y_w_user, y_w_item, bias_user, bias_item,
           global_bias):
  ne = NCHP_E * CB
  rg = _pad_edges(rate_edge_index[1], ne).reshape(NCHP_E, CR, CC)
  rs = _pad_edges(rate_edge_index[0], ne).reshape(NCHP_E, CR, CC)
  tg = _pad_edges(trust_edge_index[0], ne).reshape(NCHP_E, CR, CC)
  ts = _pad_edges(trust_edge_index[1], ne).reshape(NCHP_E, CR, CC)
  ywi2 = y_w_item.reshape(2 * NU, DH)
  ywu2 = y_w_user.reshape(2 * NU, DH)
  pqu2 = p_q_user.reshape(2 * NU, DH)
  sum_y, deg_ru, deg_ri = _edge_nodot(rg, rs, ywi2)
  sum_w, deg_ti, deg_to, lp, _db = _edge_dot(tg, ts, ywu2, pqu2)
  ue, ie, reg, link = _dense(
      sum_y[0], sum_y[1], sum_w[0], sum_w[1],
      p_q_user, p_q_item, y_w_item, y_w_user,
      bias_user, bias_item,
      deg_ru.reshape(NU, 1), deg_ri.reshape(NU, 1),
      deg_ti.reshape(NU, 1), deg_to.reshape(NU, 1),
      lp.reshape(NC * NS, 2 * L), global_bias.reshape(1, 1))
  npp = NCHP_P * CRP * CC
  cu = jnp.clip(
      _pad_edges(jnp.concatenate([pos_edge_index[0], neg_edge_index[0]]),
                 npp), 0, NU - 1).reshape(NCHP_P, CRP, CC)
  ci = jnp.clip(
      _pad_edges(jnp.concatenate([pos_edge_index[1], neg_edge_index[1]]),
                 npp), 0, NU - 1).reshape(NCHP_P, CRP, CC)
  sc = _score(cu, ci, ue, ie)
  sc = jax.tree.leaves(sc)[0].reshape(npp, 1)
  return sc[:E_PRED], sc[E_PRED:2 * E_PRED], reg[0, 0], link[0, 0]


# speedup vs baseline: 1.1134x; 1.1134x over previous
import jax
import jax.numpy as jnp
from jax import lax
from jax.experimental import pallas as pl
from jax.experimental.pallas import tpu as pltpu
from jax.experimental.pallas import tpu_sc as plsc

NU = 50000
D = 64
DH = 32
E_RATE = 800000
E_TRUST = 800000
E_PRED = 200000
LAMDA = 0.5
LAMDA_T = 0.25

NC, NS, L = 2, 16, 16
HALF = NU // NC
TRASH = 512
ACC_ROWS = 25600
CB = 640
NCH_E = E_RATE // CB
KMAX_E = 80
NCHP_E = KMAX_E * NS
NCH_P = 2 * E_PRED // CB
KMAX_P = 20
NCHP_P = KMAX_P * NC * NS
EXTW = 80
WFULL = 1568
WTAIL = HALF - (NS - 1) * WFULL

_mesh = plsc.VectorSubcoreMesh(core_axis_name="c", subcore_axis_name="s",
                               num_cores=NC, num_subcores=NS)
_sc_params = pltpu.CompilerParams(use_tc_tiling_on_sc=False)


def _lanesum(v):
  iota = lax.iota(jnp.int32, L)
  for sh in (8, 4, 2, 1):
    v = v + v.at[jnp.bitwise_xor(iota, sh)].get(mode="promise_in_bounds")
  return v


def _make_edge_pass(do_dots):
  out_type = [
      jax.ShapeDtypeStruct((2, NU, DH), jnp.float32),
      jax.ShapeDtypeStruct((NU,), jnp.float32),
      jax.ShapeDtypeStruct((NU,), jnp.float32),
  ]
  scratch = [
      pltpu.VMEM((CB,), jnp.int32),
      pltpu.VMEM((CB,), jnp.int32),
      pltpu.VMEM((CB,), jnp.int32),
      pltpu.VMEM((CB,), jnp.int32),
      pltpu.VMEM((CB,), jnp.int32),
      pltpu.VMEM((CB, DH), jnp.float32),
      pltpu.VMEM((CB,), jnp.float32),
      pltpu.VMEM((64, DH), jnp.float32),
      pltpu.VMEM((400,), jnp.float32),
      pltpu.SemaphoreType.DMA,
      pltpu.SemaphoreType.DMA,
      pltpu.VMEM_SHARED((ACC_ROWS, DH), jnp.float32),
      pltpu.VMEM_SHARED((ACC_ROWS,), jnp.float32),
      pltpu.VMEM_SHARED((ACC_ROWS,), jnp.float32),
  ]
  if do_dots:
    out_type.append(jax.ShapeDtypeStruct((NC * NS, 2, L), jnp.float32))
    out_type.append(jax.ShapeDtypeStruct((NCH_E, CB), jnp.float32))
    scratch.append(pltpu.VMEM((CB, DH), jnp.float32))
    scratch.append(pltpu.VMEM((2, L), jnp.float32))
    scratch.append(pltpu.VMEM((CB,), jnp.float32))

  def body(*refs):
    if do_dots:
      (idx_g, idx_s, gtab, ptab, sums_o, ha_o, hb_o, link_o, dbuf,
       ib_g, ib_s, ib_x, lb_s, lb_g, rows, ones, zb, zv, gsem, ssem,
       acc, ha, hb, prows, sb, db) = refs
    else:
      (idx_g, idx_s, gtab, sums_o, ha_o, hb_o,
       ib_g, ib_s, ib_x, lb_s, lb_g, rows, ones, zb, zv, gsem, ssem,
       acc, ha, hb) = refs
    c = lax.axis_index("c")
    s = lax.axis_index("s")
    iota = lax.iota(jnp.int32, L)

    def zrow(r, u):
      for j in range(DH // L):
        zb[r, pl.ds(j * L, L)] = jnp.zeros((L,), jnp.float32)
      return u
    lax.fori_loop(0, 64, zrow, 0)

    def zvec(r, u):
      zv[pl.ds(r * L, L)] = jnp.zeros((L,), jnp.float32)
      return u
    lax.fori_loop(0, 400 // L, zvec, 0)

    def orow(r, u):
      ones[pl.ds(r * L, L)] = jnp.ones((L,), jnp.float32)
      return u
    lax.fori_loop(0, CB // L, orow, 0)
    if do_dots:
      sb[0, :] = jnp.zeros((L,), jnp.float32)
      sb[1, :] = jnp.zeros((L,), jnp.float32)

    for h in (0, 1):
      def zacc(r, u):
        pltpu.sync_copy(zb, acc.at[pl.ds(s * 1600 + r * 64, 64), :])
        return u
      lax.fori_loop(0, 1600 // 64, zacc, 0)
      if h == 0:
        def zhist(r, u):
          pltpu.sync_copy(zv, ha.at[pl.ds(s * 1600 + r * 400, 400)])
          pltpu.sync_copy(zv, hb.at[pl.ds(s * 1600 + r * 400, 400)])
          return u
        lax.fori_loop(0, 4, zhist, 0)
      plsc.subcore_barrier()

      def round_body(t, u):
        k = jnp.minimum(t * NS + s, NCHP_E - 1)
        pltpu.sync_copy(idx_g.at[k], ib_g)
        pltpu.sync_copy(idx_s.at[k], ib_s)
        for j in range(CB // L):
          sl = pl.ds(j * L, L)
          tb = lax.rem(t * CB + j * L + s * L, jnp.int32(TRASH))
          tv = HALF + tb + iota
          sv = ib_s[sl]
          lvs = sv - c * HALF
          oks = (lvs >= 0) & (lvs < HALF)
          lb_s[sl] = jnp.where(oks, lvs, tv)
          gv = ib_g[sl]
          ib_x[sl] = jnp.minimum(gv, NU - 1) * 2 + h
          if h == 0:
            lvg = gv - c * HALF
            okg = (lvg >= 0) & (lvg < HALF)
            lb_g[sl] = jnp.where(okg, lvg, tv)
        gd = pltpu.async_copy(gtab.at[ib_x], rows, gsem)
        hd = []
        if h == 0:
          hd.append(pltpu.async_copy(ones, ha.at[lb_s], ssem, add=True))
          hd.append(pltpu.async_copy(ones, hb.at[lb_g], ssem, add=True))
        gd.wait()
        sd = pltpu.async_copy(rows, acc.at[lb_s], ssem, add=True)

        if do_dots:
          duty = (lax.rem(t, jnp.int32(2)) == c) & (k < NCH_E)

          @pl.when(duty)
          def _():
            for j in range(CB // L):
              sl = pl.ds(j * L, L)
              ib_x[sl] = ib_s[sl] * 2 + h
            if h == 1:
              pltpu.sync_copy(dbuf.at[k], db)
            pltpu.sync_copy(ptab.at[ib_x], prows)

            def g_body(g, u2):
              dvec = jnp.zeros((L,), jnp.float32)
              for ee in range(L):
                e = g * L + ee
                pr = jnp.zeros((L,), jnp.float32)
                for j in range(DH // L):
                  pr = pr + (rows[e, pl.ds(j * L, L)]
                             * prows[e, pl.ds(j * L, L)])
                d = _lanesum(pr)
                dvec = jnp.where(iota == ee, d, dvec)
              sl = pl.ds(g * L, L)
              if h == 0:
                db[sl] = dvec
              else:
                tot = db[sl] + dvec
                sb[0, :] = sb[0, :] + tot
                sb[1, :] = sb[1, :] + tot * tot
              return u2
            lax.fori_loop(0, CB // L, g_body, 0)
            if h == 0:
              pltpu.sync_copy(db, dbuf.at[k])

        sd.wait()
        for d_ in hd:
          d_.wait()
        return u

      lax.fori_loop(0, KMAX_E, round_body, 0)
      plsc.subcore_barrier()

      @pl.when(s < NS - 1)
      def _():
        pltpu.sync_copy(acc.at[pl.ds(s * WFULL, WFULL), :],
                        sums_o.at[h, pl.ds(c * HALF + s * WFULL, WFULL), :])
        if h == 0:
          pltpu.sync_copy(ha.at[pl.ds(s * WFULL, WFULL)],
                          ha_o.at[pl.ds(c * HALF + s * WFULL, WFULL)])
          pltpu.sync_copy(hb.at[pl.ds(s * WFULL, WFULL)],
                          hb_o.at[pl.ds(c * HALF + s * WFULL, WFULL)])

      @pl.when(s == NS - 1)
      def _():
        base = (NS - 1) * WFULL
        pltpu.sync_copy(acc.at[pl.ds(base, WTAIL), :],
                        sums_o.at[h, pl.ds(c * HALF + base, WTAIL), :])
        if h == 0:
          pltpu.sync_copy(ha.at[pl.ds(base, WTAIL)],
                          ha_o.at[pl.ds(c * HALF + base, WTAIL)])
          pltpu.sync_copy(hb.at[pl.ds(base, WTAIL)],
                          hb_o.at[pl.ds(c * HALF + base, WTAIL)])
      plsc.subcore_barrier()

    if do_dots:
      pltpu.sync_copy(sb, link_o.at[c * NS + s])

  return pl.kernel(body, out_type=tuple(out_type), mesh=_mesh,
                   compiler_params=_sc_params, scratch_types=tuple(scratch))


_edge_nodot = _make_edge_pass(False)
_edge_dot = _make_edge_pass(True)


def _score_body(idx_u, idx_i, uext, iext, out,
                ib_u, ib_i, urows, irows, scoreb, gsem):
  c = lax.axis_index("c")
  s = lax.axis_index("s")
  wid = c * NS + s
  iota = lax.iota(jnp.int32, L)

  def round_body(t, u):
    k = t * NC * NS + wid
    pltpu.sync_copy(idx_u.at[k], ib_u)
    pltpu.sync_copy(idx_i.at[k], ib_i)
    du = pltpu.async_copy(uext.at[ib_u], urows, gsem)
    di = pltpu.async_copy(iext.at[ib_i], irows, gsem)
    du.wait()
    di.wait()

    def gg_body(g, u2):
      outv = jnp.zeros((L,), jnp.float32)
      for ee in range(L):
        e = g * L + ee
        pr = jnp.zeros((L,), jnp.float32)
        for j in range(EXTW // L):
          pr = pr + (urows[e, pl.ds(j * L, L)]
                     * irows[e, pl.ds(j * L, L)])
        d = _lanesum(pr)
        outv = jnp.where(iota == ee, d, outv)
      scoreb[pl.ds(g * L, L)] = outv
      return u2
    lax.fori_loop(0, CB // L, gg_body, 0)
    pltpu.sync_copy(scoreb, out.at[k])
    return u
  lax.fori_loop(0, KMAX_P, round_body, 0)


_score = pl.kernel(
    _score_body,
    out_type=(jax.ShapeDtypeStruct((NCHP_P, CB), jnp.float32),),
    mesh=_mesh,
    compiler_params=_sc_params,
    scratch_types=(
        pltpu.VMEM((CB,), jnp.int32),
        pltpu.VMEM((CB,), jnp.int32),
        pltpu.VMEM((CB, EXTW), jnp.float32),
        pltpu.VMEM((CB, EXTW), jnp.float32),
        pltpu.VMEM((CB,), jnp.float32),
        pltpu.SemaphoreType.DMA,
    ))

_R = 1000
_G = NU // _R


def _dense_body(sy0, sy1, sw0, sw1, pqu, pqi, ywi, ywu, bu, bi,
                dru, dri, dti, dto, lp, gb, ue, ie, reg, link):
  i = pl.program_id(0)

  def fac(d):
    return jnp.where(d > 0, lax.rsqrt(jnp.maximum(d, 1.0)), 0.0)

  I_f = fac(dru[...])
  T_f = fac(dti[...])
  U_j = fac(dri[...])
  T_v = fac(dto[...])
  syr = jnp.concatenate([sy0[...], sy1[...]], axis=1)
  swr = jnp.concatenate([sw0[...], sw1[...]], axis=1)
  res = T_f * swr + I_f * syr + pqu[...]
  one = jnp.ones((_R, 1), jnp.float32)
  zer = jnp.zeros((_R, EXTW - D - 3), jnp.float32)
  gbv = jnp.full((_R, 1), gb[0, 0], jnp.float32)
  ue[...] = jnp.concatenate([res, bu[...], one, one, zer], axis=1)
  ie[...] = jnp.concatenate([pqi[...], one, bi[...], gbv, zer], axis=1)
  part = (LAMDA * jnp.sum(I_f * bu[...] ** 2)
          + LAMDA * jnp.sum(U_j * bi[...] ** 2)
          + jnp.sum((LAMDA * I_f + LAMDA_T * T_f)
                    * jnp.sum(pqu[...] ** 2, axis=1, keepdims=True))
          + LAMDA * jnp.sum(U_j * jnp.sum(pqi[...] ** 2, axis=1,
                                          keepdims=True))
          + LAMDA * jnp.sum(U_j * jnp.sum(ywi[...] ** 2, axis=1,
                                          keepdims=True))
          + LAMDA_T * jnp.sum(T_v * jnp.sum(ywu[...] ** 2, axis=1,
                                            keepdims=True)))
  prev = jnp.where(i == 0, jnp.zeros((1, 1), jnp.float32), reg[...])
  tot = prev + part
  reg[...] = jnp.where(i == _G - 1, tot / NU, tot)

  @pl.when(i == 0)
  def _():
    lpv = lp[...]
    sd = jnp.sum(lpv[:, :L])
    sd2 = jnp.sum(lpv[:, L:]) / L
    link[...] = jnp.full((1, 1), LAMDA_T * (sd2 - 2.0 * sd + E_TRUST)
                         / E_TRUST, jnp.float32)


_dense = pl.pallas_call(
    _dense_body,
    grid=(_G,),
    in_specs=[pl.BlockSpec((_R, DH), lambda i: (i, 0))] * 4
    + [pl.BlockSpec((_R, D), lambda i: (i, 0))] * 4
    + [pl.BlockSpec((_R, 1), lambda i: (i, 0))] * 6
    + [pl.BlockSpec((NC * NS, 2 * L), lambda i: (0, 0)),
       pl.BlockSpec((1, 1), lambda i: (0, 0))],
    out_specs=[pl.BlockSpec((_R, EXTW), lambda i: (i, 0)),
               pl.BlockSpec((_R, EXTW), lambda i: (i, 0)),
               pl.BlockSpec((1, 1), lambda i: (0, 0)),
               pl.BlockSpec((1, 1), lambda i: (0, 0))],
    out_shape=[jax.ShapeDtypeStruct((NU, EXTW), jnp.float32),
               jax.ShapeDtypeStruct((NU, EXTW), jnp.float32),
               jax.ShapeDtypeStruct((1, 1), jnp.float32),
               jax.ShapeDtypeStruct((1, 1), jnp.float32)],
)


def _pad_edges(v, total):
  return jnp.concatenate([v, jnp.full((total - v.shape[0],), NU, jnp.int32)])


def kernel(rate_edge_index, trust_edge_index, pos_edge_index, neg_edge_index,
           p_q_user, p_q_item, y_w_user, y_w_item, bias_user, bias_item,
           global_bias):
  ne = NCHP_E * CB
  rg = _pad_edges(rate_edge_index[1], ne).reshape(NCHP_E, CB)
  rs = _pad_edges(rate_edge_index[0], ne).reshape(NCHP_E, CB)
  tg = _pad_edges(trust_edge_index[0], ne).reshape(NCHP_E, CB)
  ts = _pad_edges(trust_edge_index[1], ne).reshape(NCHP_E, CB)
  ywi2 = y_w_item.reshape(2 * NU, DH)
  ywu2 = y_w_user.reshape(2 * NU, DH)
  pqu2 = p_q_user.reshape(2 * NU, DH)
  sum_y, deg_ru, deg_ri = _edge_nodot(rg, rs, ywi2)
  sum_w, deg_ti, deg_to, lp, _db = _edge_dot(tg, ts, ywu2, pqu2)
  ue, ie, reg, link = _dense(
      sum_y[0], sum_y[1], sum_w[0], sum_w[1],
      p_q_user, p_q_item, y_w_item, y_w_user,
      bias_user, bias_item,
      deg_ru.reshape(NU, 1), deg_ri.reshape(NU, 1),
      deg_ti.reshape(NU, 1), deg_to.reshape(NU, 1),
      lp.reshape(NC * NS, 2 * L), global_bias.reshape(1, 1))
  npp = NCHP_P * CB
  cu = jnp.clip(
      _pad_edges(jnp.concatenate([pos_edge_index[0], neg_edge_index[0]]),
                 npp), 0, NU - 1).reshape(NCHP_P, CB)
  ci = jnp.clip(
      _pad_edges(jnp.concatenate([pos_edge_index[1], neg_edge_index[1]]),
                 npp), 0, NU - 1).reshape(NCHP_P, CB)
  sc = _score(cu, ci, ue, ie)
  sc = jax.tree.leaves(sc)[0].reshape(npp, 1)
  return sc[:E_PRED], sc[E_PRED:2 * E_PRED], reg[0, 0], link[0, 0]


# speedup vs baseline: 1.8885x; 1.6961x over previous
import jax
import jax.numpy as jnp
from jax import lax
from jax.experimental import pallas as pl
from jax.experimental.pallas import tpu as pltpu
from jax.experimental.pallas import tpu_sc as plsc

NU = 50000
D = 64
DH = 32
E_RATE = 800000
E_TRUST = 800000
E_PRED = 200000
LAMDA = 0.5
LAMDA_T = 0.25

NC, NS, L = 2, 16, 16
HALF = NU // NC
TRASH = 512
ACC_ROWS = 25600
CR, CC = 10, 128
CB = CR * CC
NCH_E = E_RATE // CB
KMAX_E = -(-NCH_E // NS)
CRP = 5
NCH_P = 2 * E_PRED // (CRP * CC)
KMAX_P = -(-NCH_P // (NC * NS))
EXTW = 80
WFULL = 1568
WTAIL = HALF - (NS - 1) * WFULL

_mesh = plsc.VectorSubcoreMesh(core_axis_name="c", subcore_axis_name="s",
                               num_cores=NC, num_subcores=NS)
_sc_params = pltpu.CompilerParams(use_tc_tiling_on_sc=False)


def _lanesum(v):
  iota = lax.iota(jnp.int32, L)
  for sh in (8, 4, 2, 1):
    v = v + v.at[jnp.bitwise_xor(iota, sh)].get(mode="promise_in_bounds")
  return v


def _make_edge_pass(do_dots):
  out_type = [
      jax.ShapeDtypeStruct((2, NU, DH), jnp.float32),
      jax.ShapeDtypeStruct((NU,), jnp.float32),
      jax.ShapeDtypeStruct((NU,), jnp.float32),
  ]
  scratch = [
      pltpu.VMEM((CR, CC), jnp.int32),
      pltpu.VMEM((CR, CC), jnp.int32),
      pltpu.VMEM((CR, CC), jnp.int32),
      pltpu.VMEM((CR, CC), jnp.int32),
      pltpu.VMEM((CR, CC), jnp.int32),
      pltpu.VMEM((CR, CC, DH), jnp.float32),
      pltpu.VMEM((CR, CC), jnp.float32),
      pltpu.VMEM((64, DH), jnp.float32),
      pltpu.VMEM((400,), jnp.float32),
      pltpu.SemaphoreType.DMA,
      pltpu.VMEM_SHARED((ACC_ROWS, DH), jnp.float32),
      pltpu.VMEM_SHARED((ACC_ROWS,), jnp.float32),
      pltpu.VMEM_SHARED((ACC_ROWS,), jnp.float32),
  ]
  if do_dots:
    out_type.append(jax.ShapeDtypeStruct((NC * NS, 2, L), jnp.float32))
    out_type.append(jax.ShapeDtypeStruct((NCH_E, CR, CC), jnp.float32))
    scratch.append(pltpu.VMEM((CC, DH), jnp.float32))
    scratch.append(pltpu.VMEM((2, L), jnp.float32))
    scratch.append(pltpu.VMEM((CR, CC), jnp.float32))

  def body(*refs):
    if do_dots:
      (idx_g, idx_s, gtab, ptab, sums_o, ha_o, hb_o, link_o, dbuf,
       ib_g, ib_s, ib_x, lb_s, lb_g, rows, ones, zb, zv, sem,
       acc, ha, hb, prows, sb, db) = refs
    else:
      (idx_g, idx_s, gtab, sums_o, ha_o, hb_o,
       ib_g, ib_s, ib_x, lb_s, lb_g, rows, ones, zb, zv, sem,
       acc, ha, hb) = refs
    c = lax.axis_index("c")
    s = lax.axis_index("s")
    iota = lax.iota(jnp.int32, L)

    def zrow(r, u):
      for j in range(DH // L):
        zb[r, pl.ds(j * L, L)] = jnp.zeros((L,), jnp.float32)
      return u
    lax.fori_loop(0, 64, zrow, 0)

    def zvec(r, u):
      zv[pl.ds(r * L, L)] = jnp.zeros((L,), jnp.float32)
      return u
    lax.fori_loop(0, 400 // L, zvec, 0)

    def orow(r, u):
      for j in range(CC // L):
        ones[r, pl.ds(j * L, L)] = jnp.ones((L,), jnp.float32)
      return u
    lax.fori_loop(0, CR, orow, 0)
    if do_dots:
      sb[0, :] = jnp.zeros((L,), jnp.float32)
      sb[1, :] = jnp.zeros((L,), jnp.float32)

    for h in (0, 1):
      def zacc(r, u):
        pltpu.sync_copy(zb, acc.at[pl.ds(s * 1600 + r * 64, 64), :])
        return u
      lax.fori_loop(0, 1600 // 64, zacc, 0)
      if h == 0:
        def zhist(r, u):
          pltpu.sync_copy(zv, ha.at[pl.ds(s * 1600 + r * 400, 400)])
          pltpu.sync_copy(zv, hb.at[pl.ds(s * 1600 + r * 400, 400)])
          return u
        lax.fori_loop(0, 4, zhist, 0)
      plsc.subcore_barrier()

      def round_body(t, u):
        k = t * NS + s
        valid = k < NCH_E

        @pl.when(valid)
        def _():
          pltpu.sync_copy(idx_g.at[k], ib_g)
          pltpu.sync_copy(idx_s.at[k], ib_s)
          for a in range(CR):
            for j in range(CC // L):
              sl = pl.ds(j * L, L)
              tb = lax.rem(t * CC + (a * (CC // L) + j) * L + s * L,
                           jnp.int32(TRASH))
              tv = HALF + tb + iota
              sv = ib_s[a, sl]
              lvs = sv - c * HALF
              oks = (lvs >= 0) & (lvs < HALF)
              lb_s[a, sl] = jnp.where(oks, lvs, tv)
              ib_x[a, sl] = ib_g[a, sl] * 2 + h
              if h == 0:
                gv = ib_g[a, sl]
                lvg = gv - c * HALF
                okg = (lvg >= 0) & (lvg < HALF)
                lb_g[a, sl] = jnp.where(okg, lvg, tv)
          descs = [pltpu.async_copy(gtab.at[ib_x.at[a]], rows.at[a], sem)
                   for a in range(CR)]
          for d_ in descs:
            d_.wait()
          descs = [pltpu.async_copy(rows.at[a], acc.at[lb_s.at[a]], sem,
                                    add=True) for a in range(CR)]
          if h == 0:
            descs += [pltpu.async_copy(ones.at[a], ha.at[lb_s.at[a]], sem,
                                       add=True) for a in range(CR)]
            descs += [pltpu.async_copy(ones.at[a], hb.at[lb_g.at[a]], sem,
                                       add=True) for a in range(CR)]
          for d_ in descs:
            d_.wait()

        if not do_dots:
          return u
        duty = valid & (lax.rem(t, jnp.int32(2)) == c)

        @pl.when(duty)
        def _():
          for a in range(CR):
            for j in range(CC // L):
              sl = pl.ds(j * L, L)
              ib_x[a, sl] = ib_s[a, sl] * 2 + h
          if h == 1:
            pltpu.sync_copy(dbuf.at[k], db)
          for a in range(CR):
            pltpu.sync_copy(ptab.at[ib_x.at[a]], prows)

            def g_body(g, u2):
              dvec = jnp.zeros((L,), jnp.float32)
              for ee in range(L):
                e = g * L + ee
                pr = jnp.zeros((L,), jnp.float32)
                for j in range(DH // L):
                  pr = pr + (rows[a, e, pl.ds(j * L, L)]
                             * prows[e, pl.ds(j * L, L)])
                d = _lanesum(pr)
                dvec = jnp.where(iota == ee, d, dvec)
              sl = pl.ds(g * L, L)
              if h == 0:
                db[a, sl] = dvec
              else:
                tot = db[a, sl] + dvec
                sb[0, :] = sb[0, :] + tot
                sb[1, :] = sb[1, :] + tot * tot
              return u2
            lax.fori_loop(0, CC // L, g_body, 0)
          if h == 0:
            pltpu.sync_copy(db, dbuf.at[k])
        return u

      lax.fori_loop(0, KMAX_E, round_body, 0)
      plsc.subcore_barrier()

      @pl.when(s < NS - 1)
      def _():
        pltpu.sync_copy(acc.at[pl.ds(s * WFULL, WFULL), :],
                        sums_o.at[h, pl.ds(c * HALF + s * WFULL, WFULL), :])
        if h == 0:
          pltpu.sync_copy(ha.at[pl.ds(s * WFULL, WFULL)],
                          ha_o.at[pl.ds(c * HALF + s * WFULL, WFULL)])
          pltpu.sync_copy(hb.at[pl.ds(s * WFULL, WFULL)],
                          hb_o.at[pl.ds(c * HALF + s * WFULL, WFULL)])

      @pl.when(s == NS - 1)
      def _():
        base = (NS - 1) * WFULL
        pltpu.sync_copy(acc.at[pl.ds(base, WTAIL), :],
                        sums_o.at[h, pl.ds(c * HALF + base, WTAIL), :])
        if h == 0:
          pltpu.sync_copy(ha.at[pl.ds(base, WTAIL)],
                          ha_o.at[pl.ds(c * HALF + base, WTAIL)])
          pltpu.sync_copy(hb.at[pl.ds(base, WTAIL)],
                          hb_o.at[pl.ds(c * HALF + base, WTAIL)])
      plsc.subcore_barrier()

    if do_dots:
      pltpu.sync_copy(sb, link_o.at[c * NS + s])

  return pl.kernel(body, out_type=tuple(out_type), mesh=_mesh,
                   compiler_params=_sc_params, scratch_types=tuple(scratch))


_edge_nodot = _make_edge_pass(False)
_edge_dot = _make_edge_pass(True)


def _score_body(idx_u, idx_i, uext, iext, out,
                ib_u, ib_i, urows, irows, scoreb, gsem):
  c = lax.axis_index("c")
  s = lax.axis_index("s")
  wid = c * NS + s
  iota = lax.iota(jnp.int32, L)

  def round_body(t, u):
    k = t * NC * NS + wid

    @pl.when(k < NCH_P)
    def _():
      pltpu.sync_copy(idx_u.at[k], ib_u)
      pltpu.sync_copy(idx_i.at[k], ib_i)
      for a in range(CRP):
        du = pltpu.async_copy(uext.at[ib_u.at[a]], urows, gsem)
        di = pltpu.async_copy(iext.at[ib_i.at[a]], irows, gsem)
        du.wait()
        di.wait()

        def gg_body(g, u2):
          outv = jnp.zeros((L,), jnp.float32)
          for ee in range(L):
            e = g * L + ee
            pr = jnp.zeros((L,), jnp.float32)
            for j in range(EXTW // L):
              pr = pr + (urows[e, pl.ds(j * L, L)]
                         * irows[e, pl.ds(j * L, L)])
            d = _lanesum(pr)
            outv = jnp.where(iota == ee, d, outv)
          scoreb[a, pl.ds(g * L, L)] = outv
          return u2
        lax.fori_loop(0, CC // L, gg_body, 0)
      pltpu.sync_copy(scoreb, out.at[k])
    return u
  lax.fori_loop(0, KMAX_P, round_body, 0)


_score = pl.kernel(
    _score_body,
    out_type=(jax.ShapeDtypeStruct((NCH_P, CRP, CC), jnp.float32),),
    mesh=_mesh,
    compiler_params=_sc_params,
    scratch_types=(
        pltpu.VMEM((CRP, CC), jnp.int32),
        pltpu.VMEM((CRP, CC), jnp.int32),
        pltpu.VMEM((CC, EXTW), jnp.float32),
        pltpu.VMEM((CC, EXTW), jnp.float32),
        pltpu.VMEM((CRP, CC), jnp.float32),
        pltpu.SemaphoreType.DMA,
    ))

_R = 1000
_G = NU // _R


def _dense_body(sy0, sy1, sw0, sw1, pqu, pqi, ywi, ywu, bu, bi,
                dru, dri, dti, dto, lp, gb, ue, ie, reg, link):
  i = pl.program_id(0)

  def fac(d):
    return jnp.where(d > 0, lax.rsqrt(jnp.maximum(d, 1.0)), 0.0)

  I_f = fac(dru[...])
  T_f = fac(dti[...])
  U_j = fac(dri[...])
  T_v = fac(dto[...])
  syr = jnp.concatenate([sy0[...], sy1[...]], axis=1)
  swr = jnp.concatenate([sw0[...], sw1[...]], axis=1)
  res = T_f * swr + I_f * syr + pqu[...]
  one = jnp.ones((_R, 1), jnp.float32)
  zer = jnp.zeros((_R, EXTW - D - 3), jnp.float32)
  gbv = jnp.full((_R, 1), gb[0, 0], jnp.float32)
  ue[...] = jnp.concatenate([res, bu[...], one, one, zer], axis=1)
  ie[...] = jnp.concatenate([pqi[...], one, bi[...], gbv, zer], axis=1)
  part = (LAMDA * jnp.sum(I_f * bu[...] ** 2)
          + LAMDA * jnp.sum(U_j * bi[...] ** 2)
          + jnp.sum((LAMDA * I_f + LAMDA_T * T_f)
                    * jnp.sum(pqu[...] ** 2, axis=1, keepdims=True))
          + LAMDA * jnp.sum(U_j * jnp.sum(pqi[...] ** 2, axis=1,
                                          keepdims=True))
          + LAMDA * jnp.sum(U_j * jnp.sum(ywi[...] ** 2, axis=1,
                                          keepdims=True))
          + LAMDA_T * jnp.sum(T_v * jnp.sum(ywu[...] ** 2, axis=1,
                                            keepdims=True)))
  prev = jnp.where(i == 0, jnp.zeros((1, 1), jnp.float32), reg[...])
  tot = prev + part
  reg[...] = jnp.where(i == _G - 1, tot / NU, tot)

  @pl.when(i == 0)
  def _():
    lpv = lp[...]
    sd = jnp.sum(lpv[:, :L])
    sd2 = jnp.sum(lpv[:, L:]) / L
    link[...] = jnp.full((1, 1), LAMDA_T * (sd2 - 2.0 * sd + E_TRUST)
                         / E_TRUST, jnp.float32)


_dense = pl.pallas_call(
    _dense_body,
    grid=(_G,),
    in_specs=[pl.BlockSpec((_R, DH), lambda i: (i, 0))] * 4
    + [pl.BlockSpec((_R, D), lambda i: (i, 0))] * 4
    + [pl.BlockSpec((_R, 1), lambda i: (i, 0))] * 6
    + [pl.BlockSpec((NC * NS, 2 * L), lambda i: (0, 0)),
       pl.BlockSpec((1, 1), lambda i: (0, 0))],
    out_specs=[pl.BlockSpec((_R, EXTW), lambda i: (i, 0)),
               pl.BlockSpec((_R, EXTW), lambda i: (i, 0)),
               pl.BlockSpec((1, 1), lambda i: (0, 0)),
               pl.BlockSpec((1, 1), lambda i: (0, 0))],
    out_shape=[jax.ShapeDtypeStruct((NU, EXTW), jnp.float32),
               jax.ShapeDtypeStruct((NU, EXTW), jnp.float32),
               jax.ShapeDtypeStruct((1, 1), jnp.float32),
               jax.ShapeDtypeStruct((1, 1), jnp.float32)],
)


def kernel(rate_edge_index, trust_edge_index, pos_edge_index, neg_edge_index,
           p_q_user, p_q_item, y_w_user, y_w_item, bias_user, bias_item,
           global_bias):
  rg = rate_edge_index[1].reshape(NCH_E, CR, CC)
  rs = rate_edge_index[0].reshape(NCH_E, CR, CC)
  tg = trust_edge_index[0].reshape(NCH_E, CR, CC)
  ts = trust_edge_index[1].reshape(NCH_E, CR, CC)
  ywi2 = y_w_item.reshape(2 * NU, DH)
  ywu2 = y_w_user.reshape(2 * NU, DH)
  pqu2 = p_q_user.reshape(2 * NU, DH)
  sum_y, deg_ru, deg_ri = _edge_nodot(rg, rs, ywi2)
  sum_w, deg_ti, deg_to, lp, _db = _edge_dot(tg, ts, ywu2, pqu2)
  ue, ie, reg, link = _dense(
      sum_y[0], sum_y[1], sum_w[0], sum_w[1],
      p_q_user, p_q_item, y_w_item, y_w_user,
      bias_user, bias_item,
      deg_ru.reshape(NU, 1), deg_ri.reshape(NU, 1),
      deg_ti.reshape(NU, 1), deg_to.reshape(NU, 1),
      lp.reshape(NC * NS, 2 * L), global_bias.reshape(1, 1))
  cu = jnp.concatenate([pos_edge_index[0], neg_edge_index[0]]).reshape(
      NCH_P, CRP, CC)
  ci = jnp.concatenate([pos_edge_index[1], neg_edge_index[1]]).reshape(
      NCH_P, CRP, CC)
  sc = _score(cu, ci, ue, ie)
  sc = jax.tree.leaves(sc)[0].reshape(2 * E_PRED, 1)
  return sc[:E_PRED], sc[E_PRED:], reg[0, 0], link[0, 0]
